# Initial kernel scaffold; baseline (speedup 1.0000x reference)
import jax
import jax.numpy as jnp
from jax.experimental import pallas as pl


def kernel(prev_kc, curr_kc, prev_corr, kc_logits):
    raise NotImplementedError("write your pallas kernel here")


# speedup vs baseline: 9.2646x; 9.2646x over previous
import functools

import jax
import jax.numpy as jnp
from jax import lax
from jax.experimental import pallas as pl
from jax.experimental.pallas import tpu as pltpu
from jax.experimental.pallas import tpu_sc as plsc

B, T, K = 1024, 20, 100000
NC, NS, L = 2, 16, 16
NW = NC * NS
RPW = B // NW
NG = RPW // L
GE = L * T
GEP = 384
RPL = 24
NLINES = (K + RPL - 1) // RPL
NLP = 4168
KP = 100096
UPW = 128 * 8


def _sig(x):
    return 1.0 / (1.0 + jnp.exp(-x))


def _full(v):
    return jnp.full((L,), v, jnp.int32)


def _tc_sigmoid(x_ref, o_ref):
    o_ref[...] = _sig(x_ref[...])


_sig_call = pl.pallas_call(
    _tc_sigmoid,
    out_shape=jax.ShapeDtypeStruct((KP // 128, 128), jnp.float32),
)


@functools.lru_cache(maxsize=None)
def _build_recurrence():
  mesh = plsc.VectorSubcoreMesh(
      core_axis_name="c", subcore_axis_name="s", num_cores=NC, num_subcores=NS
  )

  @functools.partial(
      pl.kernel,
      out_type=(
          jax.ShapeDtypeStruct((B, T), jnp.float32),
          jax.ShapeDtypeStruct((NW * 8, 128), jnp.int32),
          jax.ShapeDtypeStruct((NW * 8, 128), jnp.float32),
      ),
      mesh=mesh,
      compiler_params=pltpu.CompilerParams(needs_layout_passes=False),
      scratch_types=[
          pltpu.VMEM((8, 128), jnp.int32),
          pltpu.VMEM((8, 128), jnp.int32),
          pltpu.VMEM((8, 128), jnp.int32),
          pltpu.VMEM((3, 128), jnp.int32),
          pltpu.VMEM((3, 128), jnp.int32),
          pltpu.VMEM((GEP, 128), jnp.float32),
          pltpu.VMEM((GEP, 128), jnp.float32),
          pltpu.VMEM((RPW, T), jnp.float32),
          pltpu.VMEM((8, 128), jnp.int32),
          pltpu.VMEM((8, 128), jnp.float32),
          pltpu.VMEM(((T - 1) * L,), jnp.int32),
          pltpu.VMEM(((T - 1) * L,), jnp.float32),
          pltpu.SemaphoreType.DMA,
      ],
  )
  def _recur(pk_hbm, ck_hbm, corr_hbm, tab_hbm,
             probs_hbm, ucols_hbm, uvals_hbm,
             pk_v, ck_v, corr_v, lpk_v, lck_v, gpk_v, gck_v,
             probs_v, ucols_v, uvals_v, hpk_v, hval_v, gsem):
      wid = lax.axis_index("s") * NC + lax.axis_index("c")
      base = wid * RPW

      pltpu.sync_copy(pk_hbm.at[pl.ds(wid * 8, 8)], pk_v)
      pltpu.sync_copy(ck_hbm.at[pl.ds(wid * 8, 8)], ck_v)
      pltpu.sync_copy(corr_hbm.at[pl.ds(wid * 8, 8)], corr_v)

      lane = lax.broadcasted_iota(jnp.int32, (L,), 0)

      def ld2(ref, flat):
          return plsc.load_gather(ref, [flat >> 7, flat & 127])

      def st2(ref, flat, x):
          plsc.store_scatter(ref, [flat >> 7, flat & 127], x)

      for g in range(NG):
          def line_body(q, _):
              e = g * GE + q * L + lane
              st2(lpk_v, q * L + lane, ld2(pk_v, e) // RPL)
              st2(lck_v, q * L + lane, ld2(ck_v, e) // RPL)
              return _

          lax.fori_loop(0, GEP // L, line_body, None)
          gds = []
          for c in range(3):
              gds.append(pltpu.async_copy(
                  tab_hbm.at[lpk_v.at[c]],
                  gpk_v.at[pl.ds(c * 128, 128)], gsem))
              gds.append(pltpu.async_copy(
                  tab_hbm.at[lck_v.at[c]],
                  gck_v.at[pl.ds(c * 128, 128)], gsem))
          for d in gds:
              d.wait()

          def pkcol(pk_val, eloc, col):
              ln = pk_val // RPL
              return plsc.load_gather(
                  gpk_v, [eloc, (pk_val - ln * RPL) * 5 + col])

          def ckcol(ck_val, eloc, col):
              ln = ck_val // RPL
              return plsc.load_gather(
                  gck_v, [eloc, (ck_val - ln * RPL) * 5 + col])

          lrow = g * L + lane
          e0 = lane * T
          ck0 = ld2(ck_v, lrow * T)
          c2 = _sig(ckcol(ck0, e0, 2))
          c3 = _sig(ckcol(ck0, e0, 3))
          cs = _sig(ckcol(ck0, e0, 4))
          plsc.store_scatter(probs_v, [lrow, _full(0)],
                             c2 * (1.0 - cs) + c3 * cs)

          def hist_scan(hi, key, default):
              def scan_body(j, acc):
                  pk_j = hpk_v[pl.ds(j * L, L)]
                  v_j = hval_v[pl.ds(j * L, L)]
                  return jnp.where(pk_j == key, v_j, acc)

              return lax.fori_loop(0, hi, scan_body, default)

          def step_body(i, _):
              eloc = lane * T + i
              eglob = lrow * T + i
              pk_i = ld2(pk_v, eglob)
              ck_i = ld2(ck_v, eglob)
              corr_i = ld2(corr_v, eglob)
              p0 = _sig(pkcol(pk_i, eloc, 0))
              p1 = _sig(pkcol(pk_i, eloc, 1))
              p2 = _sig(pkcol(pk_i, eloc, 2))
              p3 = _sig(pkcol(pk_i, eloc, 3))
              ss = hist_scan(i - 1, pk_i, _sig(pkcol(pk_i, eloc, 4)))
              corrb = corr_i == 1
              po0 = jnp.where(corrb, p2, 1.0 - p2)
              po1 = jnp.where(corrb, p3, 1.0 - p3)
              filt = po1 * ss / (po0 * (1.0 - ss) + po1 * ss)
              pred = p0 * (1.0 - filt) + (1.0 - p1) * filt
              hpk_v[pl.ds((i - 1) * L, L)] = pk_i
              hval_v[pl.ds((i - 1) * L, L)] = pred
              c2 = _sig(ckcol(ck_i, eloc, 2))
              c3 = _sig(ckcol(ck_i, eloc, 3))
              cs = hist_scan(i, ck_i, _sig(ckcol(ck_i, eloc, 4)))
              plsc.store_scatter(probs_v, [lrow, jnp.full((L,), i, jnp.int32)],
                                 c2 * (1.0 - cs) + c3 * cs)
              return _

          lax.fori_loop(1, T, step_body, None)

          def res_body(i, _):
              pk_i = hpk_v[pl.ds(i * L, L)]
              fin = hist_scan2(i + 1, pk_i, hval_v[pl.ds(i * L, L)])
              p = lrow * RPL + i
              st2(ucols_v, p, pk_i)
              st2(uvals_v, p, fin)
              return _

          def hist_scan2(lo, key, default):
              def scan_body(j, acc):
                  pk_j = hpk_v[pl.ds(j * L, L)]
                  v_j = hval_v[pl.ds(j * L, L)]
                  return jnp.where(pk_j == key, v_j, acc)

              return lax.fori_loop(lo, T - 1, scan_body, default)

          lax.fori_loop(0, T - 1, res_body, None)

      pltpu.sync_copy(probs_v, probs_hbm.at[pl.ds(base, RPW)])
      pltpu.sync_copy(ucols_v, ucols_hbm.at[pl.ds(wid * 8, 8)])
      pltpu.sync_copy(uvals_v, uvals_hbm.at[pl.ds(wid * 8, 8)])

  return _recur


@functools.lru_cache(maxsize=None)
def _build_fill():
  mesh = plsc.VectorSubcoreMesh(
      core_axis_name="c", subcore_axis_name="s", num_cores=NC, num_subcores=NS
  )

  @functools.partial(
      pl.kernel,
      out_type=jax.ShapeDtypeStruct((B * K,), jnp.float32),
      mesh=mesh,
      compiler_params=pltpu.CompilerParams(needs_layout_passes=False),
      scratch_types=[
          pltpu.VMEM((K,), jnp.float32),
          pltpu.VMEM((8, 128), jnp.int32),
          pltpu.VMEM((8, 128), jnp.float32),
          pltpu.SemaphoreType.DMA,
      ],
  )
  def _fill(init_hbm, ucols_hbm, uvals_hbm, state_hbm,
            init_v, ucols_v, uvals_v, fsem):
      wid = lax.axis_index("s") * NC + lax.axis_index("c")
      base = wid * RPW

      pltpu.sync_copy(ucols_hbm.at[pl.ds(wid * 8, 8)], ucols_v)
      pltpu.sync_copy(uvals_hbm.at[pl.ds(wid * 8, 8)], uvals_v)
      pltpu.sync_copy(init_hbm.at[pl.ds(0, K)], init_v)

      lane = lax.broadcasted_iota(jnp.int32, (L,), 0)
      mask_b = lane < (T - 1 - L)

      def ld2i(ref, flat):
          return plsc.load_gather(ref, [flat >> 7, flat & 127])

      for r in range(RPW):
          fa = r * RPL + lane
          fb = r * RPL + jnp.minimum(L + lane, T - 2)
          ca = ld2i(ucols_v, fa)
          cb = ld2i(ucols_v, fb)
          va = ld2i(uvals_v, fa)
          vb = ld2i(uvals_v, fb)
          sa = plsc.load_gather(init_v, [ca])
          sb = plsc.load_gather(init_v, [cb])
          plsc.store_scatter(init_v, [ca], va)
          plsc.store_scatter(init_v, [cb], vb, mask=mask_b)
          pltpu.async_copy(
              init_v, state_hbm.at[pl.ds((base + r) * K, K)], fsem).wait()
          plsc.store_scatter(init_v, [cb], sb, mask=mask_b)
          plsc.store_scatter(init_v, [ca], sa)

  return _fill


def kernel(prev_kc, curr_kc, prev_corr, kc_logits):
    lg = kc_logits.astype(jnp.float32)
    tab = jnp.pad(lg, ((0, NLP * RPL - K), (0, 0)))
    tab = jnp.pad(tab.reshape(NLP, RPL * 5), ((0, 0), (0, 8)))
    col4 = jnp.pad(lg[:, 4], (0, KP - K)).reshape(KP // 128, 128)
    init_flat = _sig_call(col4).reshape(KP)

    def _prep(a):
        a = a.astype(jnp.int32).reshape(NW, RPW * T)
        a = jnp.pad(a, ((0, 0), (0, 8 * 128 - RPW * T)))
        return a.reshape(NW * 8, 128)

    probs, ucols, uvals = _build_recurrence()(
        _prep(prev_kc), _prep(curr_kc), _prep(prev_corr), tab)
    state_flat = _build_fill()(init_flat, ucols, uvals)
    return probs, state_flat.reshape(B, K)


# speedup vs baseline: 12.0100x; 1.2963x over previous
import functools

import jax
import jax.numpy as jnp
from jax import lax
from jax.experimental import pallas as pl
from jax.experimental.pallas import tpu as pltpu
from jax.experimental.pallas import tpu_sc as plsc

B, T, K = 1024, 20, 100000
NC, NS, L = 2, 16, 16
NW = NC * NS
RPW = B // NW
NG = RPW // L
GE = L * T
GEP = 384
RPL = 16
NLP = K // RPL
KP = 100096
USTR = 24
CW = 6144
NBAND = RPW // 8
CHUNKS = [(q * CW, CW) for q in range(K // CW)] + [((K // CW) * CW, K % CW)]


def _sig(x):
    return 1.0 / (1.0 + jnp.exp(-x))


def _full(v):
    return jnp.full((L,), v, jnp.int32)


def _tc_sigmoid(x_ref, o_ref):
    o_ref[...] = jnp.broadcast_to(_sig(x_ref[...]), (8, K))


_sig_call = pl.pallas_call(
    _tc_sigmoid,
    out_shape=jax.ShapeDtypeStruct((8, K), jnp.float32),
)


@functools.lru_cache(maxsize=None)
def _build_recurrence():
  mesh = plsc.VectorSubcoreMesh(
      core_axis_name="c", subcore_axis_name="s", num_cores=NC, num_subcores=NS
  )

  @functools.partial(
      pl.kernel,
      out_type=(
          jax.ShapeDtypeStruct((B, T), jnp.float32),
          jax.ShapeDtypeStruct((NW * 8, 128), jnp.int32),
          jax.ShapeDtypeStruct((NW * 8, 128), jnp.float32),
      ),
      mesh=mesh,
      compiler_params=pltpu.CompilerParams(needs_layout_passes=False),
      scratch_types=[
          pltpu.VMEM((8, 128), jnp.int32),
          pltpu.VMEM((8, 128), jnp.int32),
          pltpu.VMEM((8, 128), jnp.int32),
          pltpu.VMEM((3, 128), jnp.int32),
          pltpu.VMEM((3, 128), jnp.int32),
          pltpu.VMEM((GEP, 128), jnp.float32),
          pltpu.VMEM((GEP, 128), jnp.float32),
          pltpu.VMEM((RPW, T), jnp.float32),
          pltpu.VMEM((8, 128), jnp.int32),
          pltpu.VMEM((8, 128), jnp.float32),
          pltpu.VMEM(((T - 1) * L,), jnp.int32),
          pltpu.VMEM(((T - 1) * L,), jnp.float32),
          pltpu.SemaphoreType.DMA,
      ],
  )
  def _recur(pk_hbm, ck_hbm, corr_hbm, tab_hbm,
             probs_hbm, ucols_hbm, uvals_hbm,
             pk_v, ck_v, corr_v, lpk_v, lck_v, gpk_v, gck_v,
             probs_v, ucols_v, uvals_v, hpk_v, hval_v, gsem):
      wid = lax.axis_index("s") * NC + lax.axis_index("c")
      base = wid * RPW

      pltpu.sync_copy(pk_hbm.at[pl.ds(wid * 8, 8)], pk_v)
      pltpu.sync_copy(ck_hbm.at[pl.ds(wid * 8, 8)], ck_v)
      pltpu.sync_copy(corr_hbm.at[pl.ds(wid * 8, 8)], corr_v)

      lane = lax.broadcasted_iota(jnp.int32, (L,), 0)

      def ld2(ref, flat):
          return plsc.load_gather(ref, [flat >> 7, flat & 127])

      def st2(ref, flat, x):
          plsc.store_scatter(ref, [flat >> 7, flat & 127], x)

      for g in range(NG):
          def line_body(q, _):
              e = g * GE + q * L + lane
              st2(lpk_v, q * L + lane, ld2(pk_v, e) >> 4)
              st2(lck_v, q * L + lane, ld2(ck_v, e) >> 4)
              return _

          lax.fori_loop(0, GEP // L, line_body, None)
          gds = []
          for c in range(3):
              gds.append(pltpu.async_copy(
                  tab_hbm.at[lpk_v.at[c]],
                  gpk_v.at[pl.ds(c * 128, 128)], gsem))
              gds.append(pltpu.async_copy(
                  tab_hbm.at[lck_v.at[c]],
                  gck_v.at[pl.ds(c * 128, 128)], gsem))
          for d in gds:
              d.wait()

          def pkcol(pk_val, eloc, col):
              return plsc.load_gather(
                  gpk_v, [eloc, ((pk_val & 15) << 3) + col])

          def ckcol(ck_val, eloc, col):
              return plsc.load_gather(
                  gck_v, [eloc, ((ck_val & 15) << 3) + col])

          lrow = g * L + lane
          e0 = lane * T
          ck0 = ld2(ck_v, lrow * T)
          c2 = _sig(ckcol(ck0, e0, 2))
          c3 = _sig(ckcol(ck0, e0, 3))
          cs = _sig(ckcol(ck0, e0, 4))
          plsc.store_scatter(probs_v, [lrow, _full(0)],
                             c2 * (1.0 - cs) + c3 * cs)

          def hist_scan(lo, hi, key, default):
              def scan_body(j, acc):
                  pk_j = hpk_v[pl.ds(j * L, L)]
                  v_j = hval_v[pl.ds(j * L, L)]
                  return jnp.where(pk_j == key, v_j, acc)

              return lax.fori_loop(lo, hi, scan_body, default)

          def step_body(i, _):
              eloc = lane * T + i
              eglob = lrow * T + i
              pk_i = ld2(pk_v, eglob)
              ck_i = ld2(ck_v, eglob)
              corr_i = ld2(corr_v, eglob)
              p0 = _sig(pkcol(pk_i, eloc, 0))
              p1 = _sig(pkcol(pk_i, eloc, 1))
              p2 = _sig(pkcol(pk_i, eloc, 2))
              p3 = _sig(pkcol(pk_i, eloc, 3))
              ss = hist_scan(0, i - 1, pk_i, _sig(pkcol(pk_i, eloc, 4)))
              corrb = corr_i == 1
              po0 = jnp.where(corrb, p2, 1.0 - p2)
              po1 = jnp.where(corrb, p3, 1.0 - p3)
              filt = po1 * ss / (po0 * (1.0 - ss) + po1 * ss)
              pred = p0 * (1.0 - filt) + (1.0 - p1) * filt
              hpk_v[pl.ds((i - 1) * L, L)] = pk_i
              hval_v[pl.ds((i - 1) * L, L)] = pred
              c2 = _sig(ckcol(ck_i, eloc, 2))
              c3 = _sig(ckcol(ck_i, eloc, 3))
              cs = hist_scan(0, i, ck_i, _sig(ckcol(ck_i, eloc, 4)))
              plsc.store_scatter(probs_v, [lrow, jnp.full((L,), i, jnp.int32)],
                                 c2 * (1.0 - cs) + c3 * cs)
              return _

          lax.fori_loop(1, T, step_body, None)

          def res_body(i, _):
              pk_i = hpk_v[pl.ds(i * L, L)]
              fin = hist_scan(i + 1, T - 1, pk_i, hval_v[pl.ds(i * L, L)])
              p = lrow * USTR + i
              st2(ucols_v, p, pk_i)
              st2(uvals_v, p, fin)
              return _

          lax.fori_loop(0, T - 1, res_body, None)

      pltpu.sync_copy(probs_v, probs_hbm.at[pl.ds(base, RPW)])
      pltpu.sync_copy(ucols_v, ucols_hbm.at[pl.ds(wid * 8, 8)])
      pltpu.sync_copy(uvals_v, uvals_hbm.at[pl.ds(wid * 8, 8)])

  return _recur


@functools.lru_cache(maxsize=None)
def _build_fill():
  mesh = plsc.VectorSubcoreMesh(
      core_axis_name="c", subcore_axis_name="s", num_cores=NC, num_subcores=NS
  )

  @functools.partial(
      pl.kernel,
      out_type=jax.ShapeDtypeStruct((B, K), jnp.float32),
      mesh=mesh,
      compiler_params=pltpu.CompilerParams(needs_layout_passes=False),
      scratch_types=[
          pltpu.VMEM((2, 8, CW), jnp.float32),
          pltpu.VMEM((2, 8, K % CW), jnp.float32),
          pltpu.VMEM((8, 128), jnp.int32),
          pltpu.VMEM((8, 128), jnp.float32),
          pltpu.SemaphoreType.DMA,
          pltpu.SemaphoreType.DMA,
      ],
  )
  def _fill(init8_hbm, ucols_hbm, uvals_hbm, state_hbm,
            buf_v, tail_v, ucols_v, uvals_v, rsem, wsem):
      wid = lax.axis_index("s") * NC + lax.axis_index("c")

      pltpu.sync_copy(ucols_hbm.at[pl.ds(wid * 8, 8)], ucols_v)
      pltpu.sync_copy(uvals_hbm.at[pl.ds(wid * 8, 8)], uvals_v)

      lane = lax.broadcasted_iota(jnp.int32, (L,), 0)
      mask_b = lane < (T - 1 - L)

      def ld2(ref, flat):
          return plsc.load_gather(ref, [flat >> 7, flat & 127])

      def _buf(q):
          p = q % 2
          if CHUNKS[q][1] != CW:
              return tail_v, p, "t%d" % p
          return buf_v, p, "b%d" % p

      def band_body(band, _):
          row0 = pl.multiple_of(wid * RPW + band * 8, 8)
          pending = {}

          def fire_read(q):
              dst, p, kq = _buf(q)
              if kq in pending:
                  pending.pop(kq).wait()
              c0, cw = CHUNKS[q]
              return pltpu.async_copy(
                  init8_hbm.at[pl.ds(0, 8), pl.ds(c0, cw)], dst.at[p], rsem)

          rd = fire_read(0)
          for q, (c0, cw) in enumerate(CHUNKS):
              dst, p, kq = _buf(q)
              if q + 1 < len(CHUNKS):
                  nrd = fire_read(q + 1)
              rd.wait()
              for r in range(8):
                  fa = (band * 8 + r) * USTR + lane
                  fb = (band * 8 + r) * USTR + jnp.minimum(L + lane, T - 2)
                  ca = ld2(ucols_v, fa)
                  cb = ld2(ucols_v, fb)
                  va = ld2(uvals_v, fa)
                  vb = ld2(uvals_v, fb)
                  ma = (ca >= c0) & (ca < c0 + cw)
                  mb = mask_b & (cb >= c0) & (cb < c0 + cw)
                  plsc.store_scatter(
                      dst, [_full(p), _full(r), ca - c0], va, mask=ma)
                  plsc.store_scatter(
                      dst, [_full(p), _full(r), cb - c0], vb, mask=mb)
              pending[kq] = pltpu.async_copy(
                  dst.at[p],
                  state_hbm.at[pl.ds(row0, 8), pl.ds(c0, cw)], wsem)
              if q + 1 < len(CHUNKS):
                  rd = nrd
          for d in pending.values():
              d.wait()
          return _

      lax.fori_loop(0, NBAND, band_body, None)

  return _fill


def kernel(prev_kc, curr_kc, prev_corr, kc_logits):
    lg = kc_logits.astype(jnp.float32)
    tab = jnp.pad(lg, ((0, 0), (0, 3))).reshape(NLP, 128)
    init8 = _sig_call(lg[:, 4].reshape(1, K))

    def _prep(a):
        a = a.astype(jnp.int32).reshape(NW, RPW * T)
        a = jnp.pad(a, ((0, 0), (0, 8 * 128 - RPW * T)))
        return a.reshape(NW * 8, 128)

    probs, ucols, uvals = _build_recurrence()(
        _prep(prev_kc), _prep(curr_kc), _prep(prev_corr), tab)
    state = _build_fill()(init8, ucols, uvals)
    return probs, state


# speedup vs baseline: 18.1348x; 1.5100x over previous
import functools

import jax
import jax.numpy as jnp
from jax import lax
from jax.experimental import pallas as pl
from jax.experimental.pallas import tpu as pltpu
from jax.experimental.pallas import tpu_sc as plsc

B, T, K = 1024, 20, 100000
NC, NS, L = 2, 16, 16
NW = NC * NS
RPW = B // NW
NG = RPW // L
GE = L * T
GEP = 384
RPL = 16
NLP = K // RPL
NSTR = 4
STRW = K // NSTR
SLOTS = 32
SENT = K
FCH = 384
NFCH = STRW // FCH


def _sig(x):
    return 1.0 / (1.0 + jnp.exp(-x))


def _full(v):
    return jnp.full((L,), v, jnp.int32)


KBLK = 4000


def _tc_sigmoid(x_ref, o_ref):
    o_ref[...] = jnp.broadcast_to(_sig(x_ref[...]), (KBLK, 128))


_sig_call = pl.pallas_call(
    _tc_sigmoid,
    grid=(K // KBLK,),
    in_specs=[pl.BlockSpec((KBLK, 1), lambda i: (i, 0))],
    out_specs=pl.BlockSpec((KBLK, 128), lambda i: (i, 0)),
    out_shape=jax.ShapeDtypeStruct((K, 128), jnp.float32),
)


@functools.lru_cache(maxsize=None)
def _build_recurrence():
  mesh = plsc.VectorSubcoreMesh(
      core_axis_name="c", subcore_axis_name="s", num_cores=NC, num_subcores=NS
  )

  @functools.partial(
      pl.kernel,
      out_type=(
          jax.ShapeDtypeStruct((B, T), jnp.float32),
          jax.ShapeDtypeStruct((B, SLOTS), jnp.int32),
          jax.ShapeDtypeStruct((B, SLOTS), jnp.float32),
      ),
      mesh=mesh,
      compiler_params=pltpu.CompilerParams(needs_layout_passes=False),
      scratch_types=[
          pltpu.VMEM((8, 128), jnp.int32),
          pltpu.VMEM((8, 128), jnp.int32),
          pltpu.VMEM((8, 128), jnp.int32),
          pltpu.VMEM((3, 128), jnp.int32),
          pltpu.VMEM((3, 128), jnp.int32),
          pltpu.VMEM((GEP, 128), jnp.float32),
          pltpu.VMEM((GEP, 128), jnp.float32),
          pltpu.VMEM((RPW, T), jnp.float32),
          pltpu.VMEM((RPW, SLOTS), jnp.int32),
          pltpu.VMEM((RPW, SLOTS), jnp.float32),
          pltpu.VMEM(((T - 1) * L,), jnp.int32),
          pltpu.VMEM(((T - 1) * L,), jnp.float32),
          pltpu.SemaphoreType.DMA,
      ],
  )
  def _recur(pk_hbm, ck_hbm, corr_hbm, tab_hbm,
             probs_hbm, ucols_hbm, uvals_hbm,
             pk_v, ck_v, corr_v, lpk_v, lck_v, gpk_v, gck_v,
             probs_v, ucols_v, uvals_v, hpk_v, hval_v, gsem):
      wid = lax.axis_index("s") * NC + lax.axis_index("c")
      base = wid * RPW

      pltpu.sync_copy(pk_hbm.at[pl.ds(wid * 8, 8)], pk_v)
      pltpu.sync_copy(ck_hbm.at[pl.ds(wid * 8, 8)], ck_v)
      pltpu.sync_copy(corr_hbm.at[pl.ds(wid * 8, 8)], corr_v)

      lane = lax.broadcasted_iota(jnp.int32, (L,), 0)

      def ld2(ref, flat):
          return plsc.load_gather(ref, [flat >> 7, flat & 127])

      def st2(ref, flat, x):
          plsc.store_scatter(ref, [flat >> 7, flat & 127], x)

      def sent_body(q, _):
          f = q * L + lane
          plsc.store_scatter(
              ucols_v, [f >> 5, f & (SLOTS - 1)], _full(SENT))
          return _

      lax.fori_loop(0, RPW * SLOTS // L, sent_body, None)

      for g in range(NG):
          def line_body(q, _):
              e = g * GE + q * L + lane
              st2(lpk_v, q * L + lane, ld2(pk_v, e) >> 4)
              st2(lck_v, q * L + lane, ld2(ck_v, e) >> 4)
              return _

          lax.fori_loop(0, GEP // L, line_body, None)
          gds = []
          for c in range(3):
              gds.append(pltpu.async_copy(
                  tab_hbm.at[lpk_v.at[c]],
                  gpk_v.at[pl.ds(c * 128, 128)], gsem))
              gds.append(pltpu.async_copy(
                  tab_hbm.at[lck_v.at[c]],
                  gck_v.at[pl.ds(c * 128, 128)], gsem))
          for d in gds:
              d.wait()

          def pkcol(pk_val, eloc, col):
              return plsc.load_gather(
                  gpk_v, [eloc, ((pk_val & 15) << 3) + col])

          def ckcol(ck_val, eloc, col):
              return plsc.load_gather(
                  gck_v, [eloc, ((ck_val & 15) << 3) + col])

          lrow = g * L + lane
          e0 = lane * T
          ck0 = ld2(ck_v, lrow * T)
          c2 = _sig(ckcol(ck0, e0, 2))
          c3 = _sig(ckcol(ck0, e0, 3))
          cs = _sig(ckcol(ck0, e0, 4))
          plsc.store_scatter(probs_v, [lrow, _full(0)],
                             c2 * (1.0 - cs) + c3 * cs)

          def hist_scan(lo, hi, key, default):
              def scan_body(j, acc):
                  pk_j = hpk_v[pl.ds(j * L, L)]
                  v_j = hval_v[pl.ds(j * L, L)]
                  return jnp.where(pk_j == key, v_j, acc)

              return lax.fori_loop(lo, hi, scan_body, default)

          def step_body(i, _):
              eloc = lane * T + i
              eglob = lrow * T + i
              pk_i = ld2(pk_v, eglob)
              ck_i = ld2(ck_v, eglob)
              corr_i = ld2(corr_v, eglob)
              p0 = _sig(pkcol(pk_i, eloc, 0))
              p1 = _sig(pkcol(pk_i, eloc, 1))
              p2 = _sig(pkcol(pk_i, eloc, 2))
              p3 = _sig(pkcol(pk_i, eloc, 3))
              ss = hist_scan(0, i - 1, pk_i, _sig(pkcol(pk_i, eloc, 4)))
              corrb = corr_i == 1
              po0 = jnp.where(corrb, p2, 1.0 - p2)
              po1 = jnp.where(corrb, p3, 1.0 - p3)
              filt = po1 * ss / (po0 * (1.0 - ss) + po1 * ss)
              pred = p0 * (1.0 - filt) + (1.0 - p1) * filt
              hpk_v[pl.ds((i - 1) * L, L)] = pk_i
              hval_v[pl.ds((i - 1) * L, L)] = pred
              c2 = _sig(ckcol(ck_i, eloc, 2))
              c3 = _sig(ckcol(ck_i, eloc, 3))
              cs = hist_scan(0, i, ck_i, _sig(ckcol(ck_i, eloc, 4)))
              plsc.store_scatter(probs_v, [lrow, jnp.full((L,), i, jnp.int32)],
                                 c2 * (1.0 - cs) + c3 * cs)
              return _

          lax.fori_loop(1, T, step_body, None)

          def res_body(i, _):
              pk_i = hpk_v[pl.ds(i * L, L)]
              fin = hist_scan(i + 1, T - 1, pk_i, hval_v[pl.ds(i * L, L)])
              plsc.store_scatter(ucols_v, [lrow, jnp.full((L,), i, jnp.int32)],
                                 pk_i)
              plsc.store_scatter(uvals_v, [lrow, jnp.full((L,), i, jnp.int32)],
                                 fin)
              return _

          lax.fori_loop(0, T - 1, res_body, None)

      pltpu.sync_copy(probs_v, probs_hbm.at[pl.ds(base, RPW)])
      pltpu.sync_copy(ucols_v, ucols_hbm.at[pl.ds(base, RPW)])
      pltpu.sync_copy(uvals_v, uvals_hbm.at[pl.ds(base, RPW)])

  return _recur


@functools.lru_cache(maxsize=None)
def _build_fill():
  mesh = plsc.VectorSubcoreMesh(
      core_axis_name="c", subcore_axis_name="s", num_cores=NC, num_subcores=NS
  )

  @functools.partial(
      pl.kernel,
      out_type=jax.ShapeDtypeStruct((K, B), jnp.float32),
      mesh=mesh,
      compiler_params=pltpu.CompilerParams(needs_layout_passes=False),
      scratch_types=[
          pltpu.VMEM((2, FCH, 128), jnp.float32),
          pltpu.VMEM((128, SLOTS), jnp.int32),
          pltpu.VMEM((128, SLOTS), jnp.float32),
          pltpu.SemaphoreType.DMA,
          pltpu.SemaphoreType.DMA,
      ],
  )
  def _fill(initt_hbm, ucols_hbm, uvals_hbm, state_hbm,
            buf_v, ucols_v, uvals_v, rsem, wsem):
      wid = lax.axis_index("s") * NC + lax.axis_index("c")
      bj = wid >> 2
      s = wid & 3
      sbase = s * STRW
      TAIL = STRW % FCH

      pltpu.sync_copy(ucols_hbm.at[pl.ds(bj * 128, 128)], ucols_v)
      pltpu.sync_copy(uvals_hbm.at[pl.ds(bj * 128, 128)], uvals_v)

      lane = lax.broadcasted_iota(jnp.int32, (L,), 0)

      def apply_edits(p, kc0, ch):
          def ed_body(q, _):
              f = q * L + lane
              row = f >> 5
              slot = f & (SLOTS - 1)
              kc = plsc.load_gather(ucols_v, [row, slot])
              va = plsc.load_gather(uvals_v, [row, slot])
              m = (kc >= kc0) & (kc < kc0 + ch)
              plsc.store_scatter(
                  buf_v, [jnp.full((L,), p, jnp.int32), kc - kc0, row],
                  va, mask=m)
              return _

          lax.fori_loop(0, 128 * SLOTS // L, ed_body, None)

      def rd_src(kc0, ch):
          return initt_hbm.at[pl.ds(kc0, ch), pl.ds(0, 128)]

      def wr_dst(kc0, ch):
          return state_hbm.at[pl.ds(kc0, ch), pl.ds(bj * 128, 128)]

      pltpu.async_copy(rd_src(sbase, FCH), buf_v.at[0], rsem)

      def chunk_body(q, _):
          p = q & 1
          kc0 = pl.multiple_of(sbase + q * FCH, 8)

          @pl.when(q > 0)
          def _():
              pltpu.make_async_copy(
                  rd_src(sbase, FCH), buf_v.at[1 - p], wsem).wait()

          @pl.when(q + 1 < NFCH)
          def _():
              pltpu.async_copy(
                  rd_src(pl.multiple_of(sbase + (q + 1) * FCH, 8), FCH),
                  buf_v.at[1 - p], rsem)

          pltpu.make_async_copy(rd_src(sbase, FCH), buf_v.at[p], rsem).wait()
          apply_edits(p, kc0, FCH)
          pltpu.async_copy(buf_v.at[p], wr_dst(kc0, FCH), wsem)
          return _

      lax.fori_loop(0, NFCH, chunk_body, None)

      lastp = (NFCH - 1) & 1
      pltpu.make_async_copy(rd_src(sbase, FCH), buf_v.at[lastp], wsem).wait()
      tbase = sbase + NFCH * FCH
      pltpu.sync_copy(rd_src(tbase, TAIL), buf_v.at[0, pl.ds(0, TAIL)])
      apply_edits(0, tbase, TAIL)
      pltpu.sync_copy(buf_v.at[0, pl.ds(0, TAIL)], wr_dst(tbase, TAIL))

  return _fill


def kernel(prev_kc, curr_kc, prev_corr, kc_logits):
    lg = kc_logits.astype(jnp.float32)
    tab = jnp.pad(lg, ((0, 0), (0, 3))).reshape(NLP, 128)
    initt = _sig_call(lg[:, 4].reshape(K, 1))

    def _prep(a):
        a = a.astype(jnp.int32).reshape(NW, RPW * T)
        a = jnp.pad(a, ((0, 0), (0, 8 * 128 - RPW * T)))
        return a.reshape(NW * 8, 128)

    probs, ucols, uvals = _build_recurrence()(
        _prep(prev_kc), _prep(curr_kc), _prep(prev_corr), tab)
    state_t = _build_fill()(initt, ucols, uvals)
    return probs, state_t.T
